# baseline (speedup 1.0000x reference)
import jax
import jax.numpy as jnp
from jax import lax
from jax.experimental import pallas as pl
from jax.experimental.pallas import tpu as pltpu
from jax.experimental.pallas import tpu_sc as plsc

BATCH = 4096
SEQ = 200
D = 64
L = 16

NC = 2
NS = 16
NW = NC * NS
ROWS_PER_W = BATCH // NW
NGROUPS = (SEQ + L - 1) // L


def _body(idx_hbm, table_hbm, out_hbm, idx_v, rows_v, gsem):
    wid = lax.axis_index("s") * NC + lax.axis_index("c")
    row0 = wid * ROWS_PER_W

    def row_step(i, carry):
        row = row0 + i
        pltpu.sync_copy(idx_hbm.at[row], idx_v)
        gather = pltpu.async_copy(table_hbm.at[idx_v], rows_v, gsem)

        def min_step(g, acc):
            base = jnp.minimum(g * L, SEQ - L)
            return jnp.minimum(acc, idx_v[pl.ds(base, L)])

        acc = lax.fori_loop(0, NGROUPS, min_step,
                            jnp.full((L,), jnp.iinfo(jnp.int32).max, jnp.int32))
        chunk_min = acc[0]
        for g in range(1, L):
            chunk_min = jnp.minimum(chunk_min, acc[g])

        gather.wait()

        @pl.when(chunk_min == 0)
        def _zero_fix():
            zeros = jnp.zeros((L,), jnp.float32)

            def group_step(g, carry2):
                base = jnp.minimum(g * L, SEQ - L)
                iv = idx_v[pl.ds(base, L)]
                for lane in range(L):
                    @pl.when(iv[lane] == 0)
                    def _zero_row(lane=lane):
                        r = base + lane
                        for j in range(D // L):
                            rows_v[r, pl.ds(j * L, L)] = zeros

                return carry2

            lax.fori_loop(0, NGROUPS, group_step, 0)

        pltpu.sync_copy(rows_v, out_hbm.at[row])
        return carry

    lax.fori_loop(0, ROWS_PER_W, row_step, 0)


def kernel(inputs, embedding_weights):
    mesh = plsc.VectorSubcoreMesh(core_axis_name="c", subcore_axis_name="s")
    return pl.kernel(
        _body,
        out_type=jax.ShapeDtypeStruct((BATCH, SEQ, D), jnp.float32),
        mesh=mesh,
        compiler_params=pltpu.CompilerParams(use_tc_tiling_on_sc=False),
        scratch_types=[
            pltpu.VMEM((SEQ,), jnp.int32),
            pltpu.VMEM((SEQ, D), jnp.float32),
            pltpu.SemaphoreType.DMA,
        ],
    )(inputs, embedding_weights)
